# baseline (speedup 1.0000x reference)
import functools

import jax
import jax.numpy as jnp
from jax import lax
from jax.experimental import pallas as pl
from jax.experimental.pallas import tpu as pltpu
from jax.experimental.pallas import tpu_sc as plsc

_INFO = plsc.get_sparse_core_info()
_NC = _INFO.num_cores
_NS = _INFO.num_subcores
_NW = _NC * _NS


@functools.lru_cache(maxsize=None)
def _make_emb(batch: int, seq: int, vocab: int, dim: int):
    assert batch % _NW == 0 and vocab % 2 == 0 and 2 * dim == 128
    bpw = batch // _NW
    half = 256
    while bpw % half:
        half //= 2
    nh = bpw // half
    mesh = plsc.VectorSubcoreMesh(core_axis_name="c", subcore_axis_name="s")

    @functools.partial(
        pl.kernel,
        mesh=mesh,
        out_type=jax.ShapeDtypeStruct((seq, dim, batch), jnp.float32),
        scratch_types=[
            pltpu.VMEM((seq, bpw), jnp.int32),
            pltpu.VMEM((vocab // 2, 2 * dim), jnp.float32),
            pltpu.VMEM((2, dim, half), jnp.float32),
            pltpu.SemaphoreType.DMA((2,)),
        ],
        compiler_params=pltpu.CompilerParams(
            use_tc_tiling_on_sc=True, needs_layout_passes=False
        ),
    )
    def emb(idx_hbm, table_hbm, out_hbm, idx_v, tab_v, plane_v, osem):
        wid = lax.axis_index("s") * _NC + lax.axis_index("c")
        wb = wid * bpw
        pltpu.sync_copy(idx_hbm.at[:, pl.ds(wb, bpw)], idx_v)
        pltpu.sync_copy(table_hbm, tab_v)

        def put(u):
            s, h = divmod(u, nh)
            return pltpu.make_async_copy(
                plane_v.at[u % 2],
                out_hbm.at[s, :, pl.ds(wb + h * half, half)],
                osem.at[u % 2],
            )

        for u in range(seq * nh):
            s, h = divmod(u, nh)
            if u >= 2:
                put(u - 2).wait()
            plane = plane_v.at[u % 2]

            @pl.loop(0, half // 16)
            def _bg(g):
                b0 = g * 16
                idx16 = idx_v[s, pl.ds(h * half + b0, 16)]
                row = lax.shift_right_logical(idx16, 1)
                colb = lax.shift_left(jnp.bitwise_and(idx16, 1), 6)

                @pl.loop(0, dim, unroll=16)
                def _d(d):
                    vec = plsc.load_gather(tab_v, [row, colb + d])
                    plane[d, pl.ds(b0, 16)] = vec

            put(u).start()

        for u in range(seq * nh - 2, seq * nh):
            put(u).wait()

    return emb


def kernel(inputs, table):
    b, s = inputs.shape
    vocab, dim = table.shape
    idx_t = inputs.T.astype(jnp.int32)
    tab2 = table.astype(jnp.float32).reshape(vocab // 2, 2 * dim)
    out_t = _make_emb(b, s, vocab, dim)(idx_t, tab2)
    return jnp.transpose(out_t, (2, 0, 1))


# speedup vs baseline: 1.7335x; 1.7335x over previous
import functools

import jax
import jax.numpy as jnp
from jax import lax
from jax.experimental import pallas as pl
from jax.experimental.pallas import tpu as pltpu
from jax.experimental.pallas import tpu_sc as plsc

_INFO = plsc.get_sparse_core_info()
_NC = _INFO.num_cores
_NS = _INFO.num_subcores
_NW = _NC * _NS


@functools.lru_cache(maxsize=None)
def _make_emb(batch: int, seq: int, vocab: int, dim: int):
    assert batch % _NW == 0
    bpw = batch // _NW
    half = 256
    while bpw % half:
        half //= 2
    nh = bpw // half
    mesh = plsc.VectorSubcoreMesh(core_axis_name="c", subcore_axis_name="s")

    @functools.partial(
        pl.kernel,
        mesh=mesh,
        out_type=jax.ShapeDtypeStruct((seq, dim, batch), jnp.float32),
        scratch_types=[
            pltpu.VMEM((seq, bpw), jnp.int32),
            pltpu.VMEM((vocab * dim,), jnp.float32),
            pltpu.VMEM((2, dim, half), jnp.float32),
            pltpu.SemaphoreType.DMA((2,)),
        ],
        compiler_params=pltpu.CompilerParams(
            use_tc_tiling_on_sc=True, needs_layout_passes=False
        ),
    )
    def emb(idx_hbm, table_hbm, out_hbm, idx_v, tab_v, plane_v, osem):
        wid = lax.axis_index("s") * _NC + lax.axis_index("c")
        wb = wid * bpw
        pltpu.sync_copy(idx_hbm.at[:, pl.ds(wb, bpw)], idx_v)
        pltpu.sync_copy(table_hbm, tab_v)

        def put(u):
            s, h = divmod(u, nh)
            return pltpu.make_async_copy(
                plane_v.at[u % 2],
                out_hbm.at[s, :, pl.ds(wb + h * half, half)],
                osem.at[u % 2],
            )

        for u in range(seq * nh):
            s, h = divmod(u, nh)
            if u >= 2:
                put(u - 2).wait()
            plane = plane_v.at[u % 2]

            @pl.loop(0, half // 16)
            def _bg(g):
                b0 = g * 16
                idx16 = idx_v[s, pl.ds(h * half + b0, 16)]
                addr = idx16 * dim

                @plsc.parallel_loop(0, dim, unroll=16)
                def _d(d):
                    vec = plsc.load_gather(tab_v, [addr + d])
                    plane[d, pl.ds(b0, 16)] = vec

            put(u).start()

        for u in range(seq * nh - 2, seq * nh):
            put(u).wait()

    return emb


def kernel(inputs, table):
    b, s = inputs.shape
    vocab, dim = table.shape
    idx_t = inputs.T.astype(jnp.int32)
    tab1 = table.astype(jnp.float32).reshape(-1)
    out_t = _make_emb(b, s, vocab, dim)(idx_t, tab1)
    return jnp.transpose(out_t, (2, 0, 1))


# speedup vs baseline: 5.7018x; 3.2891x over previous
import functools

import jax
import jax.numpy as jnp
from jax import lax
from jax.experimental import pallas as pl
from jax.experimental.pallas import tpu as pltpu
from jax.experimental.pallas import tpu_sc as plsc

_INFO = plsc.get_sparse_core_info()
_NC = _INFO.num_cores
_NS = _INFO.num_subcores
_NW = _NC * _NS


@functools.lru_cache(maxsize=None)
def _make_emb(batch: int, seq: int, vocab: int, dim: int):
    assert batch % _NW == 0
    bpw = batch // _NW
    half = 256
    while bpw % half:
        half //= 2
    nh = bpw // half
    mesh = plsc.VectorSubcoreMesh(core_axis_name="c", subcore_axis_name="s")

    @functools.partial(
        pl.kernel,
        mesh=mesh,
        out_type=jax.ShapeDtypeStruct((seq, dim, batch), jnp.float32),
        scratch_types=[
            pltpu.VMEM((seq, bpw), jnp.int32),
            pltpu.VMEM((vocab * (dim + 1),), jnp.float32),
            pltpu.VMEM((2, dim, half), jnp.float32),
            pltpu.SemaphoreType.DMA((2,)),
        ],
        compiler_params=pltpu.CompilerParams(
            use_tc_tiling_on_sc=True, needs_layout_passes=False
        ),
    )
    def emb(idx_hbm, table_hbm, out_hbm, idx_v, tab_v, plane_v, osem):
        wid = lax.axis_index("s") * _NC + lax.axis_index("c")
        wb = wid * bpw
        pltpu.sync_copy(idx_hbm.at[:, pl.ds(wb, bpw)], idx_v)
        pltpu.sync_copy(table_hbm, tab_v)

        def put(u):
            s, h = divmod(u, nh)
            return pltpu.make_async_copy(
                plane_v.at[u % 2],
                out_hbm.at[s, :, pl.ds(wb + h * half, half)],
                osem.at[u % 2],
            )

        for u in range(seq * nh):
            s, h = divmod(u, nh)
            if u >= 2:
                put(u - 2).wait()
            plane = plane_v.at[u % 2]

            @pl.loop(0, half // 16)
            def _bg(g):
                b0 = g * 16
                idx16 = idx_v[s, pl.ds(h * half + b0, 16)]
                addr = idx16 * (dim + 1)

                @plsc.parallel_loop(0, dim, unroll=16)
                def _d(d):
                    vec = plsc.load_gather(tab_v, [addr + d])
                    plane[d, pl.ds(b0, 16)] = vec

            put(u).start()

        for u in range(seq * nh - 2, seq * nh):
            put(u).wait()

    return emb


def kernel(inputs, table):
    b, s = inputs.shape
    vocab, dim = table.shape
    idx_t = inputs.T.astype(jnp.int32)
    tab1 = jnp.pad(table.astype(jnp.float32), ((0, 0), (0, 1))).reshape(-1)
    out_t = _make_emb(b, s, vocab, dim)(idx_t, tab1)
    return jnp.transpose(out_t, (2, 0, 1))


# speedup vs baseline: 5.8786x; 1.0310x over previous
import functools

import jax
import jax.numpy as jnp
from jax import lax
from jax.experimental import pallas as pl
from jax.experimental.pallas import tpu as pltpu
from jax.experimental.pallas import tpu_sc as plsc

_INFO = plsc.get_sparse_core_info()
_NC = _INFO.num_cores
_NS = _INFO.num_subcores
_NW = _NC * _NS


@functools.lru_cache(maxsize=None)
def _make_emb(batch: int, seq: int, vocab: int, dim: int):
    assert batch % _NW == 0
    bpw = batch // _NW
    half = 256
    while bpw % half:
        half //= 2
    nh = bpw // half
    mesh = plsc.VectorSubcoreMesh(core_axis_name="c", subcore_axis_name="s")

    @functools.partial(
        pl.kernel,
        mesh=mesh,
        out_type=jax.ShapeDtypeStruct((seq, dim, batch), jnp.float32),
        scratch_types=[
            pltpu.VMEM((seq, bpw), jnp.int32),
            pltpu.VMEM((vocab * (dim + 1),), jnp.float32),
            pltpu.VMEM((3, dim, half), jnp.float32),
            pltpu.SemaphoreType.DMA((3,)),
        ],
        compiler_params=pltpu.CompilerParams(
            use_tc_tiling_on_sc=True, needs_layout_passes=False
        ),
    )
    def emb(idx_hbm, table_hbm, out_hbm, idx_v, tab_v, plane_v, osem):
        wid = lax.axis_index("s") * _NC + lax.axis_index("c")
        wb = wid * bpw
        pltpu.sync_copy(idx_hbm.at[:, pl.ds(wb, bpw)], idx_v)
        pltpu.sync_copy(table_hbm, tab_v)

        def put(u):
            s, h = divmod(u, nh)
            return pltpu.make_async_copy(
                plane_v.at[u % 3],
                out_hbm.at[s, :, pl.ds(wb + h * half, half)],
                osem.at[u % 3],
            )

        for u in range(seq * nh):
            s, h = divmod(u, nh)
            if u >= 3:
                put(u - 3).wait()
            plane = plane_v.at[u % 3]

            @pl.loop(0, half // 16)
            def _bg(g):
                b0 = g * 16
                idx16 = idx_v[s, pl.ds(h * half + b0, 16)]
                addr = idx16 * (dim + 1)

                @plsc.parallel_loop(0, dim, unroll=8)
                def _d(d):
                    vec = plsc.load_gather(tab_v, [addr + d])
                    plane[d, pl.ds(b0, 16)] = vec

            put(u).start()

        for u in range(max(0, seq * nh - 3), seq * nh):
            put(u).wait()

    return emb


def kernel(inputs, table):
    b, s = inputs.shape
    vocab, dim = table.shape
    idx_t = inputs.T.astype(jnp.int32)
    tab1 = jnp.pad(table.astype(jnp.float32), ((0, 0), (0, 1))).reshape(-1)
    out_t = _make_emb(b, s, vocab, dim)(idx_t, tab1)
    return jnp.transpose(out_t, (2, 0, 1))
